# baseline (speedup 1.0000x reference)
import jax
import jax.numpy as jnp
from jax import lax
from jax.experimental import pallas as pl
from jax.experimental.pallas import tpu as pltpu
from jax.experimental.pallas import tpu_sc as plsc

B = 1024
L = 200
N = B * L
DW = 300
DP = 5
DOUT = DW + 2 * DP
DWP = 304
PTAB = 2 * 200 * 5

NC = 2
NS = 16
NW = NC * NS
ROWS_PER_TILE = N // NW
BLK = 64
NBLK = ROWS_PER_TILE // BLK
LANES = 16
OBW = BLK * DOUT


def _assemble(wordbuf, outbuf, ph_all, pt_all, p1v, p2v, off):
    lane = lax.iota(jnp.int32, LANES)

    def row_copy(r, carry2):
        wbase = r * DOUT
        for col in range(0, DWP, LANES):
            v = wordbuf[r, pl.ds(col, LANES)]
            plsc.store_scatter(outbuf, [wbase + col + lane], v)
        return carry2

    lax.fori_loop(0, BLK, row_copy, 0)

    for i in range(BLK // LANES):
        rows = lane + i * LANES
        wrow = rows * DOUT
        ph = ph_all[pl.ds(off + i * LANES, LANES)] * DP
        pt = pt_all[pl.ds(off + i * LANES, LANES)] * DP
        for j in range(DP):
            v1 = plsc.load_gather(p1v, [ph + j])
            plsc.store_scatter(outbuf, [wrow + (DW + j)], v1)
            v2 = plsc.load_gather(p2v, [pt + j])
            plsc.store_scatter(outbuf, [wrow + (DW + DP + j)], v2)


def _body(word_hbm, posh_hbm, post_hbm, wtab_hbm, p1_hbm, p2_hbm, out_hbm,
          widx, ph_all, pt_all, p1v, p2v, wb0, wb1, ob0, ob1,
          gsem0, gsem1, wsem0, wsem1):
    wid = lax.axis_index("s") * NC + lax.axis_index("c")
    tile_base = wid * ROWS_PER_TILE
    tile_wbase = tile_base * DOUT

    pltpu.sync_copy(p1_hbm, p1v)
    pltpu.sync_copy(p2_hbm, p2v)
    pltpu.sync_copy(word_hbm.at[pl.ds(tile_base, ROWS_PER_TILE)], widx)
    pltpu.sync_copy(posh_hbm.at[pl.ds(tile_base, ROWS_PER_TILE)], ph_all)
    pltpu.sync_copy(post_hbm.at[pl.ds(tile_base, ROWS_PER_TILE)], pt_all)

    wbs = (wb0, wb1)
    obs = (ob0, ob1)
    gsems = (gsem0, gsem1)
    wsems = (wsem0, wsem1)

    pltpu.async_copy(wtab_hbm.at[widx.at[pl.ds(0, BLK)]], wb0, gsem0)

    def pair(k, carry):
        for half in (0, 1):
            g = 2 * k + half

            @pl.when(g + 1 < NBLK)
            def _prefetch():
                pltpu.async_copy(
                    wtab_hbm.at[widx.at[pl.ds((g + 1) * BLK, BLK)]],
                    wbs[1 - half], gsems[1 - half])

            pltpu.make_async_copy(
                wtab_hbm.at[widx.at[pl.ds(g * BLK, BLK)]],
                wbs[half], gsems[half]).wait()

            @pl.when(g >= 2)
            def _drain_prev_write():
                pltpu.make_async_copy(
                    obs[half],
                    out_hbm.at[pl.ds(tile_wbase + (g - 2) * OBW, OBW)],
                    wsems[half]).wait()

            _assemble(wbs[half], obs[half], ph_all, pt_all, p1v, p2v, g * BLK)
            pltpu.async_copy(
                obs[half], out_hbm.at[pl.ds(tile_wbase + g * OBW, OBW)],
                wsems[half])
        return carry

    lax.fori_loop(0, NBLK // 2, pair, 0)

    pltpu.make_async_copy(
        ob0, out_hbm.at[pl.ds(tile_wbase + (NBLK - 2) * OBW, OBW)],
        wsem0).wait()
    pltpu.make_async_copy(
        ob1, out_hbm.at[pl.ds(tile_wbase + (NBLK - 1) * OBW, OBW)],
        wsem1).wait()


@jax.jit
def _run(word_flat, posh_flat, post_flat, word_table, p1_flat, p2_flat):
    mesh = plsc.VectorSubcoreMesh(
        core_axis_name="c", subcore_axis_name="s",
        num_cores=NC, num_subcores=NS)
    return pl.kernel(
        _body,
        out_type=jax.ShapeDtypeStruct((N * DOUT,), jnp.float32),
        mesh=mesh,
        compiler_params=pltpu.CompilerParams(
            use_tc_tiling_on_sc=False, needs_layout_passes=False),
        scratch_types=[
            pltpu.VMEM((ROWS_PER_TILE,), jnp.int32),
            pltpu.VMEM((ROWS_PER_TILE,), jnp.int32),
            pltpu.VMEM((ROWS_PER_TILE,), jnp.int32),
            pltpu.VMEM((PTAB,), jnp.float32),
            pltpu.VMEM((PTAB,), jnp.float32),
            pltpu.VMEM((BLK, DWP), jnp.float32),
            pltpu.VMEM((BLK, DWP), jnp.float32),
            pltpu.VMEM((OBW,), jnp.float32),
            pltpu.VMEM((OBW,), jnp.float32),
            pltpu.SemaphoreType.DMA,
            pltpu.SemaphoreType.DMA,
            pltpu.SemaphoreType.DMA,
            pltpu.SemaphoreType.DMA,
        ],
    )(word_flat, posh_flat, post_flat, word_table, p1_flat, p2_flat)


def kernel(word, posh, post, word_table, pos1_table, pos2_table):
    wf = word.reshape(N).astype(jnp.int32)
    ph = posh.reshape(N).astype(jnp.int32)
    pt = post.reshape(N).astype(jnp.int32)
    wt = jnp.pad(word_table, ((0, 0), (0, DWP - DW)))
    p1 = pos1_table.reshape(PTAB)
    p2 = pos2_table.reshape(PTAB)
    out = _run(wf, ph, pt, wt, p1, p2)
    one = p1[0] * 0.0 + 1.0
    return out.reshape(B, L, DOUT) * one


# speedup vs baseline: 1.3466x; 1.3466x over previous
import jax
import jax.numpy as jnp
from jax import lax
from jax.experimental import pallas as pl
from jax.experimental.pallas import tpu as pltpu
from jax.experimental.pallas import tpu_sc as plsc

B = 1024
L = 200
N = B * L
DW = 300
DP = 5
DOUT = DW + 2 * DP
DWP = 304
DPAD = 384
PTAB = 2 * 200 * 5

NC = 2
NS = 16
NW = NC * NS
ROWS_PER_TILE = N // NW
BLK = 64
NBLK = ROWS_PER_TILE // BLK
LANES = 16
GRP = BLK // 8
OBW = BLK * DPAD
TILE2 = 2048


def _zero_pad_lanes(outbuf):
    lane = lax.iota(jnp.int32, LANES)
    zero = jnp.zeros((LANES,), jnp.float32)
    for i in range(BLK // LANES):
        rows = lane + i * LANES
        tbase = (rows >> 3) * (3 * 1024) + TILE2 + (rows & 7) * 128
        for cm in range(48, 128, LANES):
            plsc.store_scatter(outbuf, [tbase + cm], zero)


def _assemble(wordbuf, outbuf, ph_all, pt_all, p1v, p2v, off):
    lane = lax.iota(jnp.int32, LANES)

    def row_copy(r, carry2):
        tbase = (r >> 3) * (3 * 1024) + (r & 7) * 128
        for k in range(DWP // LANES):
            col = k * LANES
            p0 = tbase + (col // 128) * 1024 + (col % 128)
            v = wordbuf[r, pl.ds(col, LANES)]
            plsc.store_scatter(outbuf, [p0 + lane], v)
        return carry2

    lax.fori_loop(0, BLK, row_copy, 0)

    for i in range(BLK // LANES):
        rows = lane + i * LANES
        tbase = (rows >> 3) * (3 * 1024) + TILE2 + (rows & 7) * 128 + 44
        ph = ph_all[pl.ds(off + i * LANES, LANES)] * DP
        pt = pt_all[pl.ds(off + i * LANES, LANES)] * DP
        for j in range(DP):
            v1 = plsc.load_gather(p1v, [ph + j])
            plsc.store_scatter(outbuf, [tbase + j], v1)
            v2 = plsc.load_gather(p2v, [pt + j])
            plsc.store_scatter(outbuf, [tbase + (DP + j)], v2)


def _body(word_hbm, posh_hbm, post_hbm, wtab_hbm, p1_hbm, p2_hbm, out_hbm,
          widx, ph_all, pt_all, p1v, p2v, wb0, wb1, ob0, ob1,
          gsem0, gsem1, wsem0, wsem1):
    wid = lax.axis_index("s") * NC + lax.axis_index("c")
    tile_base = wid * ROWS_PER_TILE
    tile_wbase = tile_base * DPAD

    pltpu.sync_copy(p1_hbm, p1v)
    pltpu.sync_copy(p2_hbm, p2v)
    pltpu.sync_copy(word_hbm.at[pl.ds(tile_base, ROWS_PER_TILE)], widx)
    pltpu.sync_copy(posh_hbm.at[pl.ds(tile_base, ROWS_PER_TILE)], ph_all)
    pltpu.sync_copy(post_hbm.at[pl.ds(tile_base, ROWS_PER_TILE)], pt_all)

    _zero_pad_lanes(ob0)
    _zero_pad_lanes(ob1)

    wbs = (wb0, wb1)
    obs = (ob0, ob1)
    gsems = (gsem0, gsem1)
    wsems = (wsem0, wsem1)

    pltpu.async_copy(wtab_hbm.at[widx.at[pl.ds(0, BLK)]], wb0, gsem0)

    def pair(k, carry):
        for half in (0, 1):
            g = 2 * k + half

            @pl.when(g + 1 < NBLK)
            def _prefetch():
                pltpu.async_copy(
                    wtab_hbm.at[widx.at[pl.ds((g + 1) * BLK, BLK)]],
                    wbs[1 - half], gsems[1 - half])

            pltpu.make_async_copy(
                wtab_hbm.at[widx.at[pl.ds(g * BLK, BLK)]],
                wbs[half], gsems[half]).wait()

            @pl.when(g >= 2)
            def _drain_prev_write():
                pltpu.make_async_copy(
                    obs[half],
                    out_hbm.at[pl.ds(tile_wbase + (g - 2) * OBW, OBW)],
                    wsems[half]).wait()

            _assemble(wbs[half], obs[half], ph_all, pt_all, p1v, p2v, g * BLK)
            pltpu.async_copy(
                obs[half], out_hbm.at[pl.ds(tile_wbase + g * OBW, OBW)],
                wsems[half])
        return carry

    lax.fori_loop(0, NBLK // 2, pair, 0)

    pltpu.make_async_copy(
        ob0, out_hbm.at[pl.ds(tile_wbase + (NBLK - 2) * OBW, OBW)],
        wsem0).wait()
    pltpu.make_async_copy(
        ob1, out_hbm.at[pl.ds(tile_wbase + (NBLK - 1) * OBW, OBW)],
        wsem1).wait()


@jax.jit
def _run(word_flat, posh_flat, post_flat, word_table, p1_flat, p2_flat):
    mesh = plsc.VectorSubcoreMesh(
        core_axis_name="c", subcore_axis_name="s",
        num_cores=NC, num_subcores=NS)
    return pl.kernel(
        _body,
        out_type=jax.ShapeDtypeStruct((N * DPAD,), jnp.float32),
        mesh=mesh,
        compiler_params=pltpu.CompilerParams(
            use_tc_tiling_on_sc=False, needs_layout_passes=False),
        scratch_types=[
            pltpu.VMEM((ROWS_PER_TILE,), jnp.int32),
            pltpu.VMEM((ROWS_PER_TILE,), jnp.int32),
            pltpu.VMEM((ROWS_PER_TILE,), jnp.int32),
            pltpu.VMEM((PTAB,), jnp.float32),
            pltpu.VMEM((PTAB,), jnp.float32),
            pltpu.VMEM((BLK, DWP), jnp.float32),
            pltpu.VMEM((BLK, DWP), jnp.float32),
            pltpu.VMEM((OBW,), jnp.float32),
            pltpu.VMEM((OBW,), jnp.float32),
            pltpu.SemaphoreType.DMA,
            pltpu.SemaphoreType.DMA,
            pltpu.SemaphoreType.DMA,
            pltpu.SemaphoreType.DMA,
        ],
    )(word_flat, posh_flat, post_flat, word_table, p1_flat, p2_flat)


def kernel(word, posh, post, word_table, pos1_table, pos2_table):
    wf = word.reshape(N).astype(jnp.int32)
    ph = posh.reshape(N).astype(jnp.int32)
    pt = post.reshape(N).astype(jnp.int32)
    wt = jnp.pad(word_table, ((0, 0), (0, DWP - DW)))
    p1 = pos1_table.reshape(PTAB)
    p2 = pos2_table.reshape(PTAB)
    out = _run(wf, ph, pt, wt, p1, p2)
    x = out.reshape(N // 8, 3, 8, 128)
    x = jnp.transpose(x, (0, 2, 1, 3))
    x = x.reshape(N, DPAD)[:, :DOUT]
    return x.reshape(B, L, DOUT)
